# baseline (speedup 1.0000x reference)
import dataclasses
import functools

import jax
import jax.numpy as jnp
from jax.experimental import pallas as pl
from jax.experimental.pallas import tpu as pltpu
from jax.experimental.pallas import tpu_sc as plsc

_BM = 4096
_BIG = 3.0e38
_LANES = 16


def _assign_body(emb_ref, cen_ref, lab_ref, asn_ref, lbl_ref):
    emb = emb_ref[...]
    cen = cen_ref[...]

    cen_sq_h = 0.5 * jnp.sum(cen * cen, axis=1, keepdims=True)
    ct = jax.lax.dot_general(
        cen, emb,
        dimension_numbers=(((1,), (1,)), ((), ())),
        preferred_element_type=jnp.float32,
    )
    d = cen_sq_h - ct
    idx = jnp.argmin(d, axis=0).astype(jnp.int32)
    iota = jax.lax.broadcasted_iota(jnp.int32, d.shape, 0)
    lab_f = lab_ref[...].astype(jnp.float32)
    lbl_f = jnp.min(jnp.where(iota == idx[None, :], lab_f, _BIG), axis=0)
    asn_ref[0, 0, :] = idx
    lbl_ref[0, 0, :] = lbl_f.astype(jnp.int32)


def _compute_assignments(embedded, leaf_centers, leaf_labels):
    b, d = embedded.shape
    k = leaf_centers.shape[0]
    nb = b // _BM
    asn, lbl = pl.pallas_call(
        _assign_body,
        grid=(nb,),
        in_specs=[
            pl.BlockSpec((_BM, d), lambda i: (i, 0)),
            pl.BlockSpec((k, d), lambda i: (0, 0)),
            pl.BlockSpec((k, 1), lambda i: (0, 0)),
        ],
        out_specs=[
            pl.BlockSpec((1, 1, _BM), lambda i: (i, 0, 0)),
            pl.BlockSpec((1, 1, _BM), lambda i: (i, 0, 0)),
        ],
        out_shape=[
            jax.ShapeDtypeStruct((nb, 1, _BM), jnp.int32),
            jax.ShapeDtypeStruct((nb, 1, _BM), jnp.int32),
        ],
        compiler_params=pltpu.CompilerParams(
            dimension_semantics=("parallel",),
        ),
    )(embedded, leaf_centers, leaf_labels.reshape(k, 1))
    return asn.reshape(b), lbl.reshape(b)


_NC, _NS = 2, 16


def _gather_labels(leaf_labels, assignments):
    n = assignments.shape[0]
    k = leaf_labels.shape[0]
    nw = _NC * _NS
    per_w = n // nw
    mesh = plsc.VectorSubcoreMesh(core_axis_name="c", subcore_axis_name="s")
    cp = pltpu.CompilerParams()
    if "needs_layout_passes" in pltpu.CompilerParams.__dataclass_fields__:
        cp = dataclasses.replace(cp, needs_layout_passes=False)

    @functools.partial(
        pl.kernel, mesh=mesh, compiler_params=cp,
        out_type=jax.ShapeDtypeStruct((n,), jnp.int32),
        scratch_types=[
            pltpu.VMEM((k,), jnp.int32),
            pltpu.VMEM((per_w,), jnp.int32),
            pltpu.VMEM((per_w,), jnp.int32),
        ],
    )
    def sc_gather(tab_hbm, idx_hbm, out_hbm, tab_v, idx_v, out_v):
        wid = jax.lax.axis_index("s") * _NC + jax.lax.axis_index("c")
        base = wid * per_w
        pltpu.sync_copy(tab_hbm, tab_v)
        pltpu.sync_copy(idx_hbm.at[pl.ds(base, per_w)], idx_v)

        @pl.loop(0, per_w, step=_LANES)
        def _(i):
            idx16 = idx_v.at[pl.ds(i, _LANES)][...]
            out_v[pl.ds(i, _LANES)] = plsc.load_gather(tab_v, [idx16])

        pltpu.sync_copy(out_v, out_hbm.at[pl.ds(base, per_w)])

    return sc_gather(leaf_labels, assignments)


def kernel(embedded, leaf_centers, leaf_labels):
    asn, lbl = _compute_assignments(embedded, leaf_centers, leaf_labels)
    return (leaf_centers, asn, lbl)


# speedup vs baseline: 1.1104x; 1.1104x over previous
import dataclasses
import functools

import jax
import jax.numpy as jnp
from jax.experimental import pallas as pl
from jax.experimental.pallas import tpu as pltpu
from jax.experimental.pallas import tpu_sc as plsc

_BM = 4096
_BIG = 3.0e38
_LANES = 16


def _assign_body(emb_ref, cen_ref, asn_ref):
    emb = emb_ref[...]
    cen = cen_ref[...]

    cen_sq_h = 0.5 * jnp.sum(cen * cen, axis=1, keepdims=True)
    ct = jax.lax.dot_general(
        cen, emb,
        dimension_numbers=(((1,), (1,)), ((), ())),
        preferred_element_type=jnp.float32,
    )
    d = cen_sq_h - ct
    asn_ref[...] = jnp.argmin(d, axis=0).astype(jnp.int32)


def _compute_assignments(embedded, leaf_centers):
    b, d = embedded.shape
    k = leaf_centers.shape[0]
    nb = b // _BM
    asn = pl.pallas_call(
        _assign_body,
        grid=(nb,),
        in_specs=[
            pl.BlockSpec((_BM, d), lambda i: (i, 0)),
            pl.BlockSpec((k, d), lambda i: (0, 0)),
        ],
        out_specs=pl.BlockSpec((_BM,), lambda i: (i,)),
        out_shape=jax.ShapeDtypeStruct((b,), jnp.int32),
        compiler_params=pltpu.CompilerParams(
            dimension_semantics=("parallel",),
        ),
    )(embedded, leaf_centers)
    return asn


_NC, _NS = 2, 16


def _gather_labels(leaf_labels, assignments):
    n = assignments.shape[0]
    k = leaf_labels.shape[0]
    nw = _NC * _NS
    per_w = n // nw
    mesh = plsc.VectorSubcoreMesh(core_axis_name="c", subcore_axis_name="s")
    cp = pltpu.CompilerParams()
    if "needs_layout_passes" in pltpu.CompilerParams.__dataclass_fields__:
        cp = dataclasses.replace(cp, needs_layout_passes=False)

    @functools.partial(
        pl.kernel, mesh=mesh, compiler_params=cp,
        out_type=jax.ShapeDtypeStruct((n,), jnp.int32),
        scratch_types=[
            pltpu.VMEM((k,), jnp.int32),
            pltpu.VMEM((per_w,), jnp.int32),
            pltpu.VMEM((per_w,), jnp.int32),
        ],
    )
    def sc_gather(tab_hbm, idx_hbm, out_hbm, tab_v, idx_v, out_v):
        wid = jax.lax.axis_index("s") * _NC + jax.lax.axis_index("c")
        base = wid * per_w
        pltpu.sync_copy(tab_hbm, tab_v)
        pltpu.sync_copy(idx_hbm.at[pl.ds(base, per_w)], idx_v)

        @pl.loop(0, per_w, step=_LANES)
        def _(i):
            idx16 = idx_v.at[pl.ds(i, _LANES)][...]
            out_v[pl.ds(i, _LANES)] = plsc.load_gather(tab_v, [idx16])

        pltpu.sync_copy(out_v, out_hbm.at[pl.ds(base, per_w)])

    return sc_gather(leaf_labels, assignments)


def kernel(embedded, leaf_centers, leaf_labels):
    asn = _compute_assignments(embedded, leaf_centers)
    lbl = _gather_labels(leaf_labels, asn)
    return (leaf_centers, asn, lbl)


# speedup vs baseline: 1.5067x; 1.3568x over previous
import dataclasses
import functools

import jax
import jax.numpy as jnp
from jax.experimental import pallas as pl
from jax.experimental.pallas import tpu as pltpu
from jax.experimental.pallas import tpu_sc as plsc

_BM = 4096
_BIG = 3.0e38
_LANES = 16


def _assign_body(embt_ref, cen_ref, asn_ref):
    embt = embt_ref[...]
    cen = cen_ref[...]

    cen_sq_h = 0.5 * jnp.sum(cen * cen, axis=1, keepdims=True)
    ct = jax.lax.dot_general(
        cen, embt,
        dimension_numbers=(((1,), (0,)), ((), ())),
        preferred_element_type=jnp.float32,
    )
    d = cen_sq_h - ct
    asn_ref[...] = jnp.argmin(d, axis=0).astype(jnp.int32)


def _compute_assignments(embedded, leaf_centers):
    embt = embedded.T
    d, b = embt.shape
    k = leaf_centers.shape[0]
    nb = b // _BM
    asn = pl.pallas_call(
        _assign_body,
        grid=(nb,),
        in_specs=[
            pl.BlockSpec((d, _BM), lambda i: (0, i)),
            pl.BlockSpec((k, d), lambda i: (0, 0)),
        ],
        out_specs=pl.BlockSpec((_BM,), lambda i: (i,)),
        out_shape=jax.ShapeDtypeStruct((b,), jnp.int32),
        compiler_params=pltpu.CompilerParams(
            dimension_semantics=("parallel",),
        ),
    )(embt, leaf_centers)
    return asn


_NC, _NS = 2, 16


def _gather_labels(leaf_labels, assignments):
    n = assignments.shape[0]
    k = leaf_labels.shape[0]
    nw = _NC * _NS
    per_w = n // nw
    mesh = plsc.VectorSubcoreMesh(core_axis_name="c", subcore_axis_name="s")
    cp = pltpu.CompilerParams()
    if "needs_layout_passes" in pltpu.CompilerParams.__dataclass_fields__:
        cp = dataclasses.replace(cp, needs_layout_passes=False)

    @functools.partial(
        pl.kernel, mesh=mesh, compiler_params=cp,
        out_type=jax.ShapeDtypeStruct((n,), jnp.int32),
        scratch_types=[
            pltpu.VMEM((k,), jnp.int32),
            pltpu.VMEM((per_w,), jnp.int32),
            pltpu.VMEM((per_w,), jnp.int32),
        ],
    )
    def sc_gather(tab_hbm, idx_hbm, out_hbm, tab_v, idx_v, out_v):
        wid = jax.lax.axis_index("s") * _NC + jax.lax.axis_index("c")
        base = wid * per_w
        pltpu.sync_copy(tab_hbm, tab_v)
        pltpu.sync_copy(idx_hbm.at[pl.ds(base, per_w)], idx_v)

        @pl.loop(0, per_w, step=_LANES)
        def _(i):
            idx16 = idx_v.at[pl.ds(i, _LANES)][...]
            out_v[pl.ds(i, _LANES)] = plsc.load_gather(tab_v, [idx16])

        pltpu.sync_copy(out_v, out_hbm.at[pl.ds(base, per_w)])

    return sc_gather(leaf_labels, assignments)


def kernel(embedded, leaf_centers, leaf_labels):
    asn = _compute_assignments(embedded, leaf_centers)
    lbl = _gather_labels(leaf_labels, asn)
    return (leaf_centers, asn, lbl)


# speedup vs baseline: 1.5386x; 1.0212x over previous
import dataclasses
import functools

import jax
import jax.numpy as jnp
from jax.experimental import pallas as pl
from jax.experimental.pallas import tpu as pltpu
from jax.experimental.pallas import tpu_sc as plsc

_BM = 8192
_BIG = 3.0e38
_LANES = 16


def _assign_body(embt_ref, cen_ref, asn_ref):
    embt = embt_ref[...]
    cen = cen_ref[...]

    cen_sq_h = 0.5 * jnp.sum(cen * cen, axis=1, keepdims=True)
    ct = jax.lax.dot_general(
        cen, embt,
        dimension_numbers=(((1,), (0,)), ((), ())),
        preferred_element_type=jnp.float32,
    )
    d = cen_sq_h - ct
    asn_ref[...] = jnp.argmin(d, axis=0).astype(jnp.int32)


def _compute_assignments(embedded, leaf_centers):
    embt = embedded.T
    d, b = embt.shape
    k = leaf_centers.shape[0]
    nb = b // _BM
    asn = pl.pallas_call(
        _assign_body,
        grid=(nb,),
        in_specs=[
            pl.BlockSpec((d, _BM), lambda i: (0, i)),
            pl.BlockSpec((k, d), lambda i: (0, 0)),
        ],
        out_specs=pl.BlockSpec((_BM,), lambda i: (i,)),
        out_shape=jax.ShapeDtypeStruct((b,), jnp.int32),
        compiler_params=pltpu.CompilerParams(
            dimension_semantics=("parallel",),
        ),
    )(embt, leaf_centers)
    return asn


_NC, _NS = 2, 16


def _gather_labels(leaf_labels, assignments):
    n = assignments.shape[0]
    k = leaf_labels.shape[0]
    nw = _NC * _NS
    per_w = n // nw
    mesh = plsc.VectorSubcoreMesh(core_axis_name="c", subcore_axis_name="s")
    cp = pltpu.CompilerParams()
    if "needs_layout_passes" in pltpu.CompilerParams.__dataclass_fields__:
        cp = dataclasses.replace(cp, needs_layout_passes=False)

    @functools.partial(
        pl.kernel, mesh=mesh, compiler_params=cp,
        out_type=jax.ShapeDtypeStruct((n,), jnp.int32),
        scratch_types=[
            pltpu.VMEM((k,), jnp.int32),
            pltpu.VMEM((per_w,), jnp.int32),
            pltpu.VMEM((per_w,), jnp.int32),
        ],
    )
    def sc_gather(tab_hbm, idx_hbm, out_hbm, tab_v, idx_v, out_v):
        wid = jax.lax.axis_index("s") * _NC + jax.lax.axis_index("c")
        base = wid * per_w
        pltpu.sync_copy(tab_hbm, tab_v)
        pltpu.sync_copy(idx_hbm.at[pl.ds(base, per_w)], idx_v)

        @pl.loop(0, per_w, step=_LANES)
        def _(i):
            idx16 = idx_v.at[pl.ds(i, _LANES)][...]
            out_v[pl.ds(i, _LANES)] = plsc.load_gather(tab_v, [idx16])

        pltpu.sync_copy(out_v, out_hbm.at[pl.ds(base, per_w)])

    return sc_gather(leaf_labels, assignments)


def kernel(embedded, leaf_centers, leaf_labels):
    asn = _compute_assignments(embedded, leaf_centers)
    lbl = _gather_labels(leaf_labels, asn)
    return (leaf_centers, asn, lbl)
